# baseline (speedup 1.0000x reference)
import functools

import jax
import jax.numpy as jnp
from jax import lax
from jax.experimental import pallas as pl
from jax.experimental.pallas import tpu as pltpu
from jax.experimental.pallas import tpu_sc as plsc

N_CLS = 600
N_CTX = 5
D = 512
SEQ = 77
SUF = SEQ - 1 - N_CTX
B = 1024

NC = 2
NS = 16
NW = NC * NS
BPW = B // NW
LANES = 16
CHUNKS = D // LANES

GRP = SUF // 2
IPW = SUF * BPW

_mesh = plsc.VectorSubcoreMesh(
    core_axis_name="c", subcore_axis_name="s", num_cores=NC, num_subcores=NS
)


@functools.partial(
    pl.kernel,
    out_type=jax.ShapeDtypeStruct((SEQ * B, D), jnp.float32),
    mesh=_mesh,
    scratch_types=[
        pltpu.VMEM((BPW,), jnp.int32),
        pltpu.VMEM((IPW,), jnp.int32),
        pltpu.VMEM((BPW, D), jnp.float32),
        pltpu.VMEM((N_CTX, D), jnp.float32),
        pltpu.VMEM((BPW, D), jnp.float32),
        pltpu.VMEM((BPW, D), jnp.float32),
        pltpu.VMEM((2 * BPW, D), jnp.float32),
        pltpu.VMEM((2 * BPW, D), jnp.float32),
        pltpu.SemaphoreType.DMA,
        pltpu.SemaphoreType.DMA,
        pltpu.SemaphoreType.DMA,
        pltpu.SemaphoreType.DMA,
    ],
)
def _prompt_kernel(
    bias_hbm, target_hbm, idxall_hbm, ctx_hbm, prefix_hbm, suffix_hbm, out_hbm,
    idx_v, idxall_v, bias_v, ctx_v, h0_v, h1_v, g0_v, g1_v,
    gsem, osem, psem, hsem,
):
    hbufs = (h0_v, h1_v)
    gbufs = (g0_v, g1_v)
    wid = lax.axis_index("s") * NC + lax.axis_index("c")
    base = wid * BPW

    pltpu.sync_copy(target_hbm.at[pl.ds(base, BPW)], idx_v)
    pltpu.sync_copy(idxall_hbm.at[pl.ds(wid * IPW, IPW)], idxall_v)
    pltpu.sync_copy(bias_hbm.at[pl.ds(base, BPW)], bias_v)
    pltpu.sync_copy(ctx_hbm, ctx_v)

    def fire_gather_group(g, bf):
        pltpu.async_copy(
            suffix_hbm.at[idxall_v.at[pl.ds(g * 2 * BPW, 2 * BPW)]],
            gbufs[bf],
            gsem,
        )

    def fire_out(plane, buf, sem):
        pltpu.async_copy(buf, out_hbm.at[pl.ds(plane * B + base, BPW)], sem)

    def fire_group_outs(g, bf):
        fire_out(1 + N_CTX + 2 * g, gbufs[bf].at[pl.ds(0, BPW)], osem)
        fire_out(1 + N_CTX + 2 * g + 1, gbufs[bf].at[pl.ds(BPW, BPW)], osem)

    def drain(sem):
        pltpu.make_async_copy(
            h0_v, out_hbm.at[pl.ds(base, BPW)], sem
        ).wait()

    def drain_gather_group():
        pltpu.make_async_copy(
            gbufs[0], out_hbm.at[pl.ds(base, 2 * BPW)], gsem
        ).wait()

    fire_gather_group(0, 0)
    fire_gather_group(1, 1)
    pcopy = pltpu.async_copy(prefix_hbm.at[idx_v], h0_v, psem)

    def compute_ctx_plane(j, hbuf):
        def chunk(c, carry):
            o = c * LANES
            cc = ctx_v[j, pl.ds(o, LANES)]

            def row(i, carry2):
                hbuf[i, pl.ds(o, LANES)] = bias_v[i, pl.ds(o, LANES)] + cc
                return carry2

            lax.fori_loop(0, BPW, row, 0, unroll=4)
            return carry

        lax.fori_loop(0, CHUNKS, chunk, 0)

    for step in range(GRP + 1):
        g = step
        if 2 <= g < GRP:
            drain(osem)
            drain(osem)
            fire_gather_group(g, g % 2)
        if step == 0:
            compute_ctx_plane(0, h1_v)
            fire_out(1, h1_v, hsem)
        elif step == 1:
            pcopy.wait()
            fire_out(0, h0_v, hsem)
        elif 2 <= step <= N_CTX:
            j = step - 1
            hbuf = hbufs[j % 2]
            drain(hsem)
            compute_ctx_plane(j, hbuf)
            fire_out(1 + j, hbuf, hsem)
        if step >= 1:
            gg = step - 1
            drain_gather_group()
            fire_group_outs(gg, gg % 2)

    drain(osem)
    drain(osem)
    g70 = pltpu.async_copy(
        suffix_hbm.at[idxall_v.at[pl.ds(2 * GRP * BPW, BPW)]],
        g1_v.at[pl.ds(0, BPW)],
        gsem,
    )
    g70.wait()
    fire_out(1 + N_CTX + SUF - 1, g1_v.at[pl.ds(0, BPW)], osem)

    drain(osem)
    drain(osem)
    drain(osem)
    drain(hsem)
    drain(hsem)


def kernel(bias, target, ctx, token_prefix, token_suffix):
    target = target.astype(jnp.int32)
    prefix2 = token_prefix.reshape(N_CLS, D)
    suffix2 = jnp.transpose(token_suffix, (1, 0, 2)).reshape(SUF * N_CLS, D)
    plane_off = jnp.arange(SUF, dtype=jnp.int32) * N_CLS
    idx_all = (
        target.reshape(NW, 1, BPW) + plane_off[None, :, None]
    ).reshape(NW * IPW)
    out2 = _prompt_kernel(bias, target, idx_all, ctx, prefix2, suffix2)
    return jnp.transpose(out2.reshape(SEQ, B, D), (1, 0, 2))
